# Initial kernel scaffold; baseline (speedup 1.0000x reference)
import jax
import jax.numpy as jnp
from jax.experimental import pallas as pl


def kernel(R, Z, atom_mol_batch, edge_index, params):
    raise NotImplementedError("write your pallas kernel here")


# speedup vs baseline: 3.8536x; 3.8536x over previous
import functools
import math

import jax
import jax.numpy as jnp
from jax import lax
from jax.experimental import pallas as pl
from jax.experimental.pallas import tpu as pltpu
from jax.experimental.pallas import tpu_sc as plsc

N_ATOMS = 10000
N_EDGES = 320000
F_DIM = 128
K_RBF = 64
N_MOL = 256
N_EMB = 95
RC = 10.0
LOG2 = math.log(2.0)

NC = 2
NS = 16
NW = NC * NS
L = 16

EPT = N_EDGES // NW
CH = 80
NCHUNK = EPT // CH
RPT = 624
ZCH = 208
TAIL = N_ATOMS - NS * RPT

_mesh = plsc.VectorSubcoreMesh(core_axis_name="c", subcore_axis_name="s")
_sc_params = pltpu.CompilerParams(needs_layout_passes=False)


@functools.partial(
    pl.kernel,
    out_type=jax.ShapeDtypeStruct((N_EDGES,), jnp.float32),
    mesh=_mesh,
    scratch_types=[
        pltpu.VMEM((N_ATOMS,), jnp.float32),
        pltpu.VMEM((N_ATOMS,), jnp.float32),
        pltpu.VMEM((N_ATOMS,), jnp.float32),
        pltpu.VMEM((EPT,), jnp.int32),
        pltpu.VMEM((EPT,), jnp.int32),
        pltpu.VMEM((EPT,), jnp.float32),
    ],
    compiler_params=_sc_params,
)
def _sc_d2(rx_hbm, ry_hbm, rz_hbm, src_hbm, dst_hbm, out_hbm,
           rx_v, ry_v, rz_v, src_v, dst_v, d2_v):
    wid = lax.axis_index("s") * NC + lax.axis_index("c")
    base = wid * EPT
    pltpu.sync_copy(rx_hbm, rx_v)
    pltpu.sync_copy(ry_hbm, ry_v)
    pltpu.sync_copy(rz_hbm, rz_v)
    pltpu.sync_copy(src_hbm.at[pl.ds(base, EPT)], src_v)
    pltpu.sync_copy(dst_hbm.at[pl.ds(base, EPT)], dst_v)

    def body(i, carry):
        s = src_v[pl.ds(i * L, L)]
        t = dst_v[pl.ds(i * L, L)]
        ex = plsc.load_gather(rx_v, [s]) - plsc.load_gather(rx_v, [t]) + 1e-6
        ey = plsc.load_gather(ry_v, [s]) - plsc.load_gather(ry_v, [t]) + 1e-6
        ez = plsc.load_gather(rz_v, [s]) - plsc.load_gather(rz_v, [t]) + 1e-6
        d2_v[pl.ds(i * L, L)] = ex * ex + ey * ey + ez * ez
        return carry

    lax.fori_loop(0, EPT // L, body, 0)
    pltpu.sync_copy(d2_v, out_hbm.at[pl.ds(base, EPT)])


@functools.partial(
    pl.kernel,
    out_type=jax.ShapeDtypeStruct((NC, N_ATOMS, F_DIM), jnp.float32),
    mesh=_mesh,
    scratch_types=[
        pltpu.VMEM_SHARED((N_ATOMS, F_DIM), jnp.float32),
        pltpu.VMEM((CH, F_DIM), jnp.float32),
        pltpu.VMEM((CH, F_DIM), jnp.float32),
        pltpu.VMEM((ZCH, F_DIM), jnp.float32),
        pltpu.VMEM((CH,), jnp.int32),
        pltpu.VMEM((CH,), jnp.int32),
        pltpu.SemaphoreType.DMA,
    ],
    compiler_params=_sc_params,
)
def _sc_gms(g_hbm, mj_hbm, src_hbm, dst_hbm, out_hbm,
            agg_sh, rows_v, g_v, zero_v, src_v, dst_v, sem):
    cid = lax.axis_index("c")
    sid = lax.axis_index("s")
    wid = sid * NC + cid
    base = wid * EPT

    def zbody(i, carry):
        z = jnp.zeros((L,), jnp.float32)
        for k in range(F_DIM // L):
            zero_v[i, pl.ds(k * L, L)] = z
        return carry

    lax.fori_loop(0, ZCH, zbody, 0)
    for j in range(RPT // ZCH):
        pltpu.sync_copy(zero_v, agg_sh.at[pl.ds(sid * RPT + j * ZCH, ZCH)])

    @pl.when(sid == NS - 1)
    def _():
        pltpu.sync_copy(zero_v.at[pl.ds(0, TAIL)],
                        agg_sh.at[pl.ds(NS * RPT, TAIL)])

    plsc.subcore_barrier()

    def body(j, carry):
        pltpu.sync_copy(src_hbm.at[pl.ds(base + j * CH, CH)], src_v)
        pltpu.sync_copy(dst_hbm.at[pl.ds(base + j * CH, CH)], dst_v)
        gat = pltpu.async_copy(mj_hbm.at[src_v], rows_v, sem)
        pltpu.sync_copy(g_hbm.at[pl.ds(base + j * CH, CH)], g_v)
        gat.wait()

        def mul_body(r, c2):
            for k in range(F_DIM // L):
                sl = pl.ds(k * L, L)
                rows_v[r, sl] = rows_v[r, sl] * g_v[r, sl]
            return c2

        lax.fori_loop(0, CH, mul_body, 0)
        pltpu.sync_copy(rows_v, agg_sh.at[dst_v], add=True)
        return carry

    lax.fori_loop(0, NCHUNK, body, 0)
    plsc.subcore_barrier()
    pltpu.sync_copy(agg_sh.at[pl.ds(sid * RPT, RPT)],
                    out_hbm.at[cid, pl.ds(sid * RPT, RPT)])

    @pl.when(sid == NS - 1)
    def _():
        pltpu.sync_copy(agg_sh.at[pl.ds(NS * RPT, TAIL)],
                        out_hbm.at[cid, pl.ds(NS * RPT, TAIL)])


def _ssp(x):
    return jnp.maximum(x, 0.0) + jnp.log1p(jnp.exp(-jnp.abs(x))) - LOG2


BE = 5000
BN = 2000


def _edge_g_body(d2_ref, c_ref, w_ref, wg0_ref, wg1_ref, g0_ref, g1_ref):
    d = jnp.sqrt(d2_ref[...])
    xr = d * (1.0 / RC)
    poly = 1.0 - 6.0 * xr**5 + 15.0 * xr**4 - 10.0 * xr**3
    phi = jnp.where(d < RC, poly, 0.0)
    expd = jnp.exp(-d)
    c = c_ref[...]
    w = w_ref[...]
    rbf = phi * jnp.exp(-w * (expd - c) ** 2)
    g0_ref[...] = jnp.dot(rbf, wg0_ref[...], preferred_element_type=jnp.float32)
    g1_ref[...] = jnp.dot(rbf, wg1_ref[...], preferred_element_type=jnp.float32)


def _edge_g(d2, centers, widths, wg0, wg1):
    ne = d2.shape[0]
    grid = (ne // BE,)
    full = lambda shape: pl.BlockSpec(shape, lambda i: (0, 0))
    return pl.pallas_call(
        _edge_g_body,
        grid=grid,
        in_specs=[
            pl.BlockSpec((BE, 1), lambda i: (i, 0)),
            full((1, K_RBF)), full((1, K_RBF)),
            full((K_RBF, F_DIM)), full((K_RBF, F_DIM)),
        ],
        out_specs=[
            pl.BlockSpec((BE, F_DIM), lambda i: (i, 0)),
            pl.BlockSpec((BE, F_DIM), lambda i: (i, 0)),
        ],
        out_shape=[
            jax.ShapeDtypeStruct((ne, F_DIM), jnp.float32),
            jax.ShapeDtypeStruct((ne, F_DIM), jnp.float32),
        ],
    )(d2, centers, widths, wg0, wg1)


def _node_first_body(z_ref, emb_ref, wj_ref, bj_ref, wi_ref, bi_ref,
                     vi_ref, mj_ref, vp_ref):
    z = z_ref[...]
    oh = (z == lax.broadcasted_iota(jnp.int32, (BN, N_EMB), 1)).astype(jnp.float32)
    vi = jnp.dot(oh, emb_ref[...], preferred_element_type=jnp.float32)
    xa = _ssp(vi)
    vi_ref[...] = vi
    mj_ref[...] = _ssp(jnp.dot(xa, wj_ref[...], preferred_element_type=jnp.float32)
                       + bj_ref[...])
    vp_ref[...] = _ssp(jnp.dot(xa, wi_ref[...], preferred_element_type=jnp.float32)
                       + bi_ref[...])


def _node_first(z, emb, wj, bj, wi, bi):
    grid = (N_ATOMS // BN,)
    full = lambda shape: pl.BlockSpec(shape, lambda i: (0, 0))
    nb = lambda w: pl.BlockSpec((BN, w), lambda i: (i, 0))
    sds = jax.ShapeDtypeStruct((N_ATOMS, F_DIM), jnp.float32)
    return pl.pallas_call(
        _node_first_body,
        grid=grid,
        in_specs=[nb(1), full((N_EMB, F_DIM)),
                  full((F_DIM, F_DIM)), full((1, F_DIM)),
                  full((F_DIM, F_DIM)), full((1, F_DIM))],
        out_specs=[nb(F_DIM), nb(F_DIM), nb(F_DIM)],
        out_shape=[sds, sds, sds],
    )(z, emb, wj, bj, wi, bi)


def _module_tail(x, v, p):
    v = v + jnp.dot(_ssp(_ssp(v) @ p["Wri1"] + p["bri1"]), p["Wri2"],
                    preferred_element_type=jnp.float32) + p["bri2"]
    xn = p["u"] * x + jnp.dot(_ssp(v), p["Wout"],
                              preferred_element_type=jnp.float32) + p["bout"]
    xn = xn + jnp.dot(_ssp(_ssp(xn) @ p["Wra1"] + p["bra1"]), p["Wra2"],
                      preferred_element_type=jnp.float32) + p["bra2"]
    t = xn + jnp.dot(_ssp(_ssp(xn) @ p["Wro1"] + p["bro1"]), p["Wro2"],
                     preferred_element_type=jnp.float32) + p["bro2"]
    out = jnp.dot(_ssp(t), p["Wo"], preferred_element_type=jnp.float32) + p["bo"]
    return xn, out


_TAIL_KEYS = ("Wri1", "bri1", "Wri2", "bri2", "u", "Wout", "bout",
              "Wra1", "bra1", "Wra2", "bra2", "Wro1", "bro1", "Wro2", "bro2",
              "Wo", "bo")


def _node_mid_body(x_ref, vp_ref, a0_ref, a1_ref, *rest):
    tail_refs = rest[:len(_TAIL_KEYS)]
    wj_ref, bj_ref, wi_ref, bi_ref = rest[len(_TAIL_KEYS):len(_TAIL_KEYS) + 4]
    x1_ref, out0_ref, mj_ref, vp1_ref = rest[len(_TAIL_KEYS) + 4:]
    p = {k: r[...] for k, r in zip(_TAIL_KEYS, tail_refs)}
    v = vp_ref[...] + a0_ref[0] + a1_ref[0]
    xn, out = _module_tail(x_ref[...], v, p)
    x1_ref[...] = xn
    out0_ref[...] = out
    xa = _ssp(xn)
    mj_ref[...] = _ssp(jnp.dot(xa, wj_ref[...], preferred_element_type=jnp.float32)
                       + bj_ref[...])
    vp1_ref[...] = _ssp(jnp.dot(xa, wi_ref[...], preferred_element_type=jnp.float32)
                        + bi_ref[...])


def _tail_specs(full):
    specs = []
    for k in _TAIL_KEYS:
        if k == "Wo":
            specs.append(full((F_DIM, 2)))
        elif k == "bo":
            specs.append(full((1, 2)))
        elif k.startswith("W"):
            specs.append(full((F_DIM, F_DIM)))
        else:
            specs.append(full((1, F_DIM)))
    return specs


def _node_mid(x, vp, aggp, tail_ws, wj, bj, wi, bi):
    grid = (N_ATOMS // BN,)
    full = lambda shape: pl.BlockSpec(shape, lambda i: (0, 0))
    nb = lambda w: pl.BlockSpec((BN, w), lambda i: (i, 0))
    a0 = pl.BlockSpec((1, BN, F_DIM), lambda i: (0, i, 0))
    a1 = pl.BlockSpec((1, BN, F_DIM), lambda i: (1, i, 0))
    sds = jax.ShapeDtypeStruct((N_ATOMS, F_DIM), jnp.float32)
    return pl.pallas_call(
        _node_mid_body,
        grid=grid,
        in_specs=[nb(F_DIM), nb(F_DIM), a0, a1] + _tail_specs(full)
                 + [full((F_DIM, F_DIM)), full((1, F_DIM)),
                    full((F_DIM, F_DIM)), full((1, F_DIM))],
        out_specs=[nb(F_DIM), nb(2), nb(F_DIM), nb(F_DIM)],
        out_shape=[sds, jax.ShapeDtypeStruct((N_ATOMS, 2), jnp.float32), sds, sds],
    )(x, vp, aggp, aggp, *tail_ws, wj, bj, wi, bi)


def _node_last_body(x_ref, vp_ref, a0_ref, a1_ref, out0_ref, z_ref, r_ref, b_ref,
                    *rest):
    tail_refs = rest[:len(_TAIL_KEYS)]
    scale_ref, shift_ref = rest[len(_TAIL_KEYS):len(_TAIL_KEYS) + 2]
    mol_ref = rest[len(_TAIL_KEYS) + 2]
    p = {k: r[...] for k, r in zip(_TAIL_KEYS, tail_refs)}
    v = vp_ref[...] + a0_ref[0] + a1_ref[0]
    _, out1 = _module_tail(x_ref[...], v, p)
    sep = out0_ref[...] + out1
    z = z_ref[...]
    oh = (z == lax.broadcasted_iota(jnp.int32, (BN, N_EMB), 1)).astype(jnp.float32)
    sc = jnp.dot(oh, scale_ref[...], preferred_element_type=jnp.float32)
    sh = jnp.dot(oh, shift_ref[...], preferred_element_type=jnp.float32)
    ao = sc * sep + sh
    vals = jnp.concatenate([ao, ao[:, 1:2] * r_ref[...]], axis=1)
    ohm = (b_ref[...] == lax.broadcasted_iota(jnp.int32, (BN, N_MOL), 1)
           ).astype(jnp.float32)
    part = lax.dot_general(ohm, vals, (((0,), (0,)), ((), ())),
                           preferred_element_type=jnp.float32)

    @pl.when(pl.program_id(0) == 0)
    def _():
        mol_ref[...] = jnp.zeros_like(mol_ref)

    mol_ref[...] += part


def _node_last(x, vp, aggp, out0, z, r, b, tail_ws, scale, shift):
    grid = (N_ATOMS // BN,)
    full = lambda shape: pl.BlockSpec(shape, lambda i: (0, 0))
    nb = lambda w: pl.BlockSpec((BN, w), lambda i: (i, 0))
    a0 = pl.BlockSpec((1, BN, F_DIM), lambda i: (0, i, 0))
    a1 = pl.BlockSpec((1, BN, F_DIM), lambda i: (1, i, 0))
    return pl.pallas_call(
        _node_last_body,
        grid=grid,
        in_specs=[nb(F_DIM), nb(F_DIM), a0, a1, nb(2), nb(1), nb(3), nb(1)]
                 + _tail_specs(full) + [full((N_EMB, 2)), full((N_EMB, 2))],
        out_specs=pl.BlockSpec((N_MOL, 5), lambda i: (0, 0)),
        out_shape=jax.ShapeDtypeStruct((N_MOL, 5), jnp.float32),
    )(x, vp, aggp, aggp, out0, z, r, b, *tail_ws, scale, shift)


def kernel(R, Z, atom_mol_batch, edge_index, params):
    src = edge_index[0].astype(jnp.int32)
    dst = edge_index[1].astype(jnp.int32)
    z2 = Z.astype(jnp.int32).reshape(N_ATOMS, 1)
    b2 = atom_mol_batch.astype(jnp.int32).reshape(N_ATOMS, 1)

    d2 = _sc_d2(R[:, 0], R[:, 1], R[:, 2], src, dst)
    g0, g1 = _edge_g(d2.reshape(N_EDGES, 1),
                     params["centers"].reshape(1, K_RBF),
                     params["widths"].reshape(1, K_RBF),
                     params["Wg"][0], params["Wg"][1])

    def mod_ws(m):
        out = []
        for k in _TAIL_KEYS:
            w = params[k][m]
            if w.ndim == 1:
                w = w.reshape(1, -1)
            out.append(w)
        return out

    vi, mj0, vp0 = _node_first(z2, params["emb"],
                               params["Wj"][0], params["bj"][0].reshape(1, F_DIM),
                               params["Wi"][0], params["bi"][0].reshape(1, F_DIM))
    aggp0 = _sc_gms(g0, mj0, src, dst)
    x1, out0, mj1, vp1 = _node_mid(vi, vp0, aggp0, mod_ws(0),
                                   params["Wj"][1], params["bj"][1].reshape(1, F_DIM),
                                   params["Wi"][1], params["bi"][1].reshape(1, F_DIM))
    aggp1 = _sc_gms(g1, mj1, src, dst)
    mol = _node_last(x1, vp1, aggp1, out0, z2, R, b2, mod_ws(1),
                     params["scale"], params["shift"])
    return mol


# speedup vs baseline: 5.5426x; 1.4383x over previous
import functools
import math

import jax
import jax.numpy as jnp
from jax import lax
from jax.experimental import pallas as pl
from jax.experimental.pallas import tpu as pltpu
from jax.experimental.pallas import tpu_sc as plsc

N_ATOMS = 10000
N_EDGES = 320000
F_DIM = 128
K_RBF = 64
N_MOL = 256
N_EMB = 95
RC = 10.0
LOG2 = math.log(2.0)

NC = 2
NS = 16
NW = NC * NS
L = 16

EPT = N_EDGES // NW
CH = 80
NCHUNK = EPT // CH
RPT = 624
ZCH = 24
TAIL = N_ATOMS - NS * RPT

_mesh = plsc.VectorSubcoreMesh(core_axis_name="c", subcore_axis_name="s")
_sc_params = pltpu.CompilerParams(needs_layout_passes=False)


@functools.partial(
    pl.kernel,
    out_type=jax.ShapeDtypeStruct((N_EDGES,), jnp.float32),
    mesh=_mesh,
    scratch_types=[
        pltpu.VMEM((N_ATOMS,), jnp.float32),
        pltpu.VMEM((N_ATOMS,), jnp.float32),
        pltpu.VMEM((N_ATOMS,), jnp.float32),
        pltpu.VMEM((EPT,), jnp.int32),
        pltpu.VMEM((EPT,), jnp.int32),
        pltpu.VMEM((EPT,), jnp.float32),
    ],
    compiler_params=_sc_params,
)
def _sc_d2(rx_hbm, ry_hbm, rz_hbm, src_hbm, dst_hbm, out_hbm,
           rx_v, ry_v, rz_v, src_v, dst_v, d2_v):
    wid = lax.axis_index("s") * NC + lax.axis_index("c")
    base = wid * EPT
    pltpu.sync_copy(rx_hbm, rx_v)
    pltpu.sync_copy(ry_hbm, ry_v)
    pltpu.sync_copy(rz_hbm, rz_v)
    pltpu.sync_copy(src_hbm.at[pl.ds(base, EPT)], src_v)
    pltpu.sync_copy(dst_hbm.at[pl.ds(base, EPT)], dst_v)

    def body(i, carry):
        s = src_v[pl.ds(i * L, L)]
        t = dst_v[pl.ds(i * L, L)]
        ex = plsc.load_gather(rx_v, [s]) - plsc.load_gather(rx_v, [t]) + 1e-6
        ey = plsc.load_gather(ry_v, [s]) - plsc.load_gather(ry_v, [t]) + 1e-6
        ez = plsc.load_gather(rz_v, [s]) - plsc.load_gather(rz_v, [t]) + 1e-6
        d2_v[pl.ds(i * L, L)] = ex * ex + ey * ey + ez * ez
        return carry

    lax.fori_loop(0, EPT // L, body, 0)
    pltpu.sync_copy(d2_v, out_hbm.at[pl.ds(base, EPT)])


@functools.partial(
    pl.kernel,
    out_type=jax.ShapeDtypeStruct((NC, N_ATOMS, F_DIM), jnp.float32),
    mesh=_mesh,
    scratch_types=[
        pltpu.VMEM_SHARED((N_ATOMS, F_DIM), jnp.float32),
        pltpu.VMEM((2, CH, F_DIM), jnp.float32),
        pltpu.VMEM((2, CH, F_DIM), jnp.float32),
        pltpu.VMEM((ZCH, F_DIM), jnp.float32),
        pltpu.VMEM((2, 2, CH), jnp.int32),
        pltpu.SemaphoreType.DMA,
        pltpu.SemaphoreType.DMA,
        pltpu.SemaphoreType.DMA,
        pltpu.SemaphoreType.DMA,
        pltpu.SemaphoreType.DMA,
        pltpu.SemaphoreType.DMA,
    ],
    compiler_params=_sc_params,
)
def _sc_gms(g_hbm, mj_hbm, eidx_hbm, out_hbm,
            agg_sh, rows_v, g_v, zero_v, pidx_v,
            sr0, sr1, sg0, sg1, si0, si1):
    cid = lax.axis_index("c")
    sid = lax.axis_index("s")
    wid = sid * NC + cid
    base = wid * EPT
    srows = (sr0, sr1)
    sgs = (sg0, sg1)
    sis = (si0, si1)

    def zbody(i, carry):
        z = jnp.zeros((L,), jnp.float32)
        for k in range(F_DIM // L):
            zero_v[i, pl.ds(k * L, L)] = z
        return carry

    lax.fori_loop(0, ZCH, zbody, 0)
    for j in range(RPT // ZCH):
        pltpu.sync_copy(zero_v, agg_sh.at[pl.ds(sid * RPT + j * ZCH, ZCH)])

    @pl.when(sid == NS - 1)
    def _():
        pltpu.sync_copy(zero_v.at[pl.ds(0, TAIL)],
                        agg_sh.at[pl.ds(NS * RPT, TAIL)])

    plsc.subcore_barrier()

    def idx_copy(c, b):
        return pltpu.async_copy(eidx_hbm.at[wid, c], pidx_v.at[b], sis[b])

    def data_copies(c, b):
        pltpu.async_copy(mj_hbm.at[pidx_v.at[b, 0]], rows_v.at[b], srows[b])
        pltpu.async_copy(g_hbm.at[pl.ds(base + c * CH, CH)], g_v.at[b], sgs[b])

    def wait_data(c, b):
        pltpu.make_async_copy(mj_hbm.at[pidx_v.at[b, 0]], rows_v.at[b],
                              srows[b]).wait()
        pltpu.make_async_copy(g_hbm.at[pl.ds(base + c * CH, CH)], g_v.at[b],
                              sgs[b]).wait()

    def mul_scatter(b):
        def mul_body(r, c2):
            for k in range(F_DIM // L):
                sl = pl.ds(k * L, L)
                rows_v[b, r, sl] = rows_v[b, r, sl] * g_v[b, r, sl]
            return c2

        lax.fori_loop(0, CH, mul_body, 0)
        pltpu.sync_copy(rows_v.at[b], agg_sh.at[pidx_v.at[b, 1]], add=True)

    idx_copy(0, 0).wait()
    data_copies(0, 0)
    idx_copy(1, 1)

    def chunk_body(c, a):
        bb = 1 - a
        wait_data(c, a)
        pltpu.make_async_copy(eidx_hbm.at[wid, c + 1], pidx_v.at[bb],
                              sis[bb]).wait()
        data_copies(c + 1, bb)
        mul_scatter(a)

        @pl.when(c + 2 < NCHUNK)
        def _():
            idx_copy(c + 2, a)

    def pair(i, carry):
        chunk_body(2 * i, 0)
        chunk_body(2 * i + 1, 1)
        return carry

    lax.fori_loop(0, (NCHUNK - 1) // 2, pair, 0)
    wait_data(NCHUNK - 1, 0)
    mul_scatter(0)
    plsc.subcore_barrier()
    pltpu.sync_copy(agg_sh.at[pl.ds(sid * RPT, RPT)],
                    out_hbm.at[cid, pl.ds(sid * RPT, RPT)])

    @pl.when(sid == NS - 1)
    def _():
        pltpu.sync_copy(agg_sh.at[pl.ds(NS * RPT, TAIL)],
                        out_hbm.at[cid, pl.ds(NS * RPT, TAIL)])


def _ssp(x):
    return jnp.maximum(x, 0.0) + jnp.log1p(jnp.exp(-jnp.abs(x))) - LOG2


BE = 5000
BN = 2000


def _edge_g_body(d2_ref, c_ref, w_ref, wg0_ref, wg1_ref, g0_ref, g1_ref):
    d = jnp.sqrt(d2_ref[...])
    xr = d * (1.0 / RC)
    poly = 1.0 - 6.0 * xr**5 + 15.0 * xr**4 - 10.0 * xr**3
    phi = jnp.where(d < RC, poly, 0.0)
    expd = jnp.exp(-d)
    c = c_ref[...]
    w = w_ref[...]
    rbf = phi * jnp.exp(-w * (expd - c) ** 2)
    g0_ref[...] = jnp.dot(rbf, wg0_ref[...], preferred_element_type=jnp.float32)
    g1_ref[...] = jnp.dot(rbf, wg1_ref[...], preferred_element_type=jnp.float32)


def _edge_g(d2, centers, widths, wg0, wg1):
    ne = d2.shape[0]
    grid = (ne // BE,)
    full = lambda shape: pl.BlockSpec(shape, lambda i: (0, 0))
    return pl.pallas_call(
        _edge_g_body,
        grid=grid,
        in_specs=[
            pl.BlockSpec((BE, 1), lambda i: (i, 0)),
            full((1, K_RBF)), full((1, K_RBF)),
            full((K_RBF, F_DIM)), full((K_RBF, F_DIM)),
        ],
        out_specs=[
            pl.BlockSpec((BE, F_DIM), lambda i: (i, 0)),
            pl.BlockSpec((BE, F_DIM), lambda i: (i, 0)),
        ],
        out_shape=[
            jax.ShapeDtypeStruct((ne, F_DIM), jnp.float32),
            jax.ShapeDtypeStruct((ne, F_DIM), jnp.float32),
        ],
    )(d2, centers, widths, wg0, wg1)


def _node_first_body(z_ref, emb_ref, wj_ref, bj_ref, wi_ref, bi_ref,
                     vi_ref, mj_ref, vp_ref):
    z = z_ref[...]
    oh = (z == lax.broadcasted_iota(jnp.int32, (BN, N_EMB), 1)).astype(jnp.float32)
    vi = jnp.dot(oh, emb_ref[...], preferred_element_type=jnp.float32)
    xa = _ssp(vi)
    vi_ref[...] = vi
    mj_ref[...] = _ssp(jnp.dot(xa, wj_ref[...], preferred_element_type=jnp.float32)
                       + bj_ref[...])
    vp_ref[...] = _ssp(jnp.dot(xa, wi_ref[...], preferred_element_type=jnp.float32)
                       + bi_ref[...])


def _node_first(z, emb, wj, bj, wi, bi):
    grid = (N_ATOMS // BN,)
    full = lambda shape: pl.BlockSpec(shape, lambda i: (0, 0))
    nb = lambda w: pl.BlockSpec((BN, w), lambda i: (i, 0))
    sds = jax.ShapeDtypeStruct((N_ATOMS, F_DIM), jnp.float32)
    return pl.pallas_call(
        _node_first_body,
        grid=grid,
        in_specs=[nb(1), full((N_EMB, F_DIM)),
                  full((F_DIM, F_DIM)), full((1, F_DIM)),
                  full((F_DIM, F_DIM)), full((1, F_DIM))],
        out_specs=[nb(F_DIM), nb(F_DIM), nb(F_DIM)],
        out_shape=[sds, sds, sds],
    )(z, emb, wj, bj, wi, bi)


def _module_tail(x, v, p):
    v = v + jnp.dot(_ssp(_ssp(v) @ p["Wri1"] + p["bri1"]), p["Wri2"],
                    preferred_element_type=jnp.float32) + p["bri2"]
    xn = p["u"] * x + jnp.dot(_ssp(v), p["Wout"],
                              preferred_element_type=jnp.float32) + p["bout"]
    xn = xn + jnp.dot(_ssp(_ssp(xn) @ p["Wra1"] + p["bra1"]), p["Wra2"],
                      preferred_element_type=jnp.float32) + p["bra2"]
    t = xn + jnp.dot(_ssp(_ssp(xn) @ p["Wro1"] + p["bro1"]), p["Wro2"],
                     preferred_element_type=jnp.float32) + p["bro2"]
    out = jnp.dot(_ssp(t), p["Wo"], preferred_element_type=jnp.float32) + p["bo"]
    return xn, out


_TAIL_KEYS = ("Wri1", "bri1", "Wri2", "bri2", "u", "Wout", "bout",
              "Wra1", "bra1", "Wra2", "bra2", "Wro1", "bro1", "Wro2", "bro2",
              "Wo", "bo")


def _node_mid_body(x_ref, vp_ref, a0_ref, a1_ref, *rest):
    tail_refs = rest[:len(_TAIL_KEYS)]
    wj_ref, bj_ref, wi_ref, bi_ref = rest[len(_TAIL_KEYS):len(_TAIL_KEYS) + 4]
    x1_ref, out0_ref, mj_ref, vp1_ref = rest[len(_TAIL_KEYS) + 4:]
    p = {k: r[...] for k, r in zip(_TAIL_KEYS, tail_refs)}
    v = vp_ref[...] + a0_ref[0] + a1_ref[0]
    xn, out = _module_tail(x_ref[...], v, p)
    x1_ref[...] = xn
    out0_ref[...] = out
    xa = _ssp(xn)
    mj_ref[...] = _ssp(jnp.dot(xa, wj_ref[...], preferred_element_type=jnp.float32)
                       + bj_ref[...])
    vp1_ref[...] = _ssp(jnp.dot(xa, wi_ref[...], preferred_element_type=jnp.float32)
                        + bi_ref[...])


def _tail_specs(full):
    specs = []
    for k in _TAIL_KEYS:
        if k == "Wo":
            specs.append(full((F_DIM, 2)))
        elif k == "bo":
            specs.append(full((1, 2)))
        elif k.startswith("W"):
            specs.append(full((F_DIM, F_DIM)))
        else:
            specs.append(full((1, F_DIM)))
    return specs


def _node_mid(x, vp, aggp, tail_ws, wj, bj, wi, bi):
    grid = (N_ATOMS // BN,)
    full = lambda shape: pl.BlockSpec(shape, lambda i: (0, 0))
    nb = lambda w: pl.BlockSpec((BN, w), lambda i: (i, 0))
    a0 = pl.BlockSpec((1, BN, F_DIM), lambda i: (0, i, 0))
    a1 = pl.BlockSpec((1, BN, F_DIM), lambda i: (1, i, 0))
    sds = jax.ShapeDtypeStruct((N_ATOMS, F_DIM), jnp.float32)
    return pl.pallas_call(
        _node_mid_body,
        grid=grid,
        in_specs=[nb(F_DIM), nb(F_DIM), a0, a1] + _tail_specs(full)
                 + [full((F_DIM, F_DIM)), full((1, F_DIM)),
                    full((F_DIM, F_DIM)), full((1, F_DIM))],
        out_specs=[nb(F_DIM), nb(2), nb(F_DIM), nb(F_DIM)],
        out_shape=[sds, jax.ShapeDtypeStruct((N_ATOMS, 2), jnp.float32), sds, sds],
    )(x, vp, aggp, aggp, *tail_ws, wj, bj, wi, bi)


def _node_last_body(x_ref, vp_ref, a0_ref, a1_ref, out0_ref, z_ref, r_ref, b_ref,
                    *rest):
    tail_refs = rest[:len(_TAIL_KEYS)]
    scale_ref, shift_ref = rest[len(_TAIL_KEYS):len(_TAIL_KEYS) + 2]
    mol_ref = rest[len(_TAIL_KEYS) + 2]
    p = {k: r[...] for k, r in zip(_TAIL_KEYS, tail_refs)}
    v = vp_ref[...] + a0_ref[0] + a1_ref[0]
    _, out1 = _module_tail(x_ref[...], v, p)
    sep = out0_ref[...] + out1
    z = z_ref[...]
    oh = (z == lax.broadcasted_iota(jnp.int32, (BN, N_EMB), 1)).astype(jnp.float32)
    sc = jnp.dot(oh, scale_ref[...], preferred_element_type=jnp.float32)
    sh = jnp.dot(oh, shift_ref[...], preferred_element_type=jnp.float32)
    ao = sc * sep + sh
    vals = jnp.concatenate([ao, ao[:, 1:2] * r_ref[...]], axis=1)
    ohm = (b_ref[...] == lax.broadcasted_iota(jnp.int32, (BN, N_MOL), 1)
           ).astype(jnp.float32)
    part = lax.dot_general(ohm, vals, (((0,), (0,)), ((), ())),
                           preferred_element_type=jnp.float32)

    @pl.when(pl.program_id(0) == 0)
    def _():
        mol_ref[...] = jnp.zeros_like(mol_ref)

    mol_ref[...] += part


def _node_last(x, vp, aggp, out0, z, r, b, tail_ws, scale, shift):
    grid = (N_ATOMS // BN,)
    full = lambda shape: pl.BlockSpec(shape, lambda i: (0, 0))
    nb = lambda w: pl.BlockSpec((BN, w), lambda i: (i, 0))
    a0 = pl.BlockSpec((1, BN, F_DIM), lambda i: (0, i, 0))
    a1 = pl.BlockSpec((1, BN, F_DIM), lambda i: (1, i, 0))
    return pl.pallas_call(
        _node_last_body,
        grid=grid,
        in_specs=[nb(F_DIM), nb(F_DIM), a0, a1, nb(2), nb(1), nb(3), nb(1)]
                 + _tail_specs(full) + [full((N_EMB, 2)), full((N_EMB, 2))],
        out_specs=pl.BlockSpec((N_MOL, 5), lambda i: (0, 0)),
        out_shape=jax.ShapeDtypeStruct((N_MOL, 5), jnp.float32),
    )(x, vp, aggp, aggp, out0, z, r, b, *tail_ws, scale, shift)


def kernel(R, Z, atom_mol_batch, edge_index, params):
    src = edge_index[0].astype(jnp.int32)
    dst = edge_index[1].astype(jnp.int32)
    eidx = jnp.stack([src.reshape(NW, NCHUNK, CH),
                      dst.reshape(NW, NCHUNK, CH)], axis=2)
    z2 = Z.astype(jnp.int32).reshape(N_ATOMS, 1)
    b2 = atom_mol_batch.astype(jnp.int32).reshape(N_ATOMS, 1)

    d2 = _sc_d2(R[:, 0], R[:, 1], R[:, 2], src, dst)
    g0, g1 = _edge_g(d2.reshape(N_EDGES, 1),
                     params["centers"].reshape(1, K_RBF),
                     params["widths"].reshape(1, K_RBF),
                     params["Wg"][0], params["Wg"][1])

    def mod_ws(m):
        out = []
        for k in _TAIL_KEYS:
            w = params[k][m]
            if w.ndim == 1:
                w = w.reshape(1, -1)
            out.append(w)
        return out

    vi, mj0, vp0 = _node_first(z2, params["emb"],
                               params["Wj"][0], params["bj"][0].reshape(1, F_DIM),
                               params["Wi"][0], params["bi"][0].reshape(1, F_DIM))
    aggp0 = _sc_gms(g0, mj0, eidx)
    x1, out0, mj1, vp1 = _node_mid(vi, vp0, aggp0, mod_ws(0),
                                   params["Wj"][1], params["bj"][1].reshape(1, F_DIM),
                                   params["Wi"][1], params["bi"][1].reshape(1, F_DIM))
    aggp1 = _sc_gms(g1, mj1, eidx)
    mol = _node_last(x1, vp1, aggp1, out0, z2, R, b2, mod_ws(1),
                     params["scale"], params["shift"])
    return mol
